# Initial kernel scaffold; baseline (speedup 1.0000x reference)
import jax
import jax.numpy as jnp
from jax.experimental import pallas as pl


def kernel(x, edge_index, W1, att_src1, att_dst1, b1, W2, att_src2, att_dst2, b2):
    raise NotImplementedError("write your pallas kernel here")


# speedup vs baseline: 21.5292x; 21.5292x over previous
import functools

import jax
import jax.numpy as jnp
from jax import lax
from jax.experimental import pallas as pl
from jax.experimental.pallas import tpu as pltpu
from jax.experimental.pallas import tpu_sc as plsc

N = 10000
E = 320000
D_IN = 128
HID = 64
H1 = 8
C2 = 2
W1AUG = 80
W2AUG = 16

NS = 16
NC = 2
NP = 10240
BN = 1024
NB = NP // BN

E1T = E // NS
E2T = E // (NC * NS)
K = 80
NCH1 = E1T // K
NCH2 = E2T // K
RPT = NP // NS


def _ka_body(x_ref, w1_ref, as_ref, ad_ref, haug_ref, asrc_ref, adst_ref):
    i = pl.program_id(0)
    h = jnp.dot(x_ref[...], w1_ref[...], preferred_element_type=jnp.float32)
    ones = jnp.ones((BN, 1), jnp.float32)
    pad = jnp.zeros((BN, W1AUG - HID - 1), jnp.float32)
    for hh in range(H1):
        hcol = h[:, hh * HID:(hh + 1) * HID]
        haug_ref[hh] = jnp.concatenate([hcol, ones, pad], axis=1)
        col = pl.ds(i * BN, BN)
        asrc_ref[pl.ds(hh, 1), col] = jnp.sum(
            hcol * as_ref[hh][None, :], axis=1)[None, :]
        adst_ref[pl.ds(hh, 1), col] = jnp.sum(
            hcol * ad_ref[hh][None, :], axis=1)[None, :]


def _stage_a(x, w1, att_src1, att_dst1):
    return pl.pallas_call(
        _ka_body,
        grid=(NB,),
        in_specs=[
            pl.BlockSpec((BN, D_IN), lambda i: (i, 0)),
            pl.BlockSpec((D_IN, H1 * HID), lambda i: (0, 0)),
            pl.BlockSpec((H1, HID), lambda i: (0, 0)),
            pl.BlockSpec((H1, HID), lambda i: (0, 0)),
        ],
        out_specs=[
            pl.BlockSpec((H1, BN, W1AUG), lambda i: (0, i, 0)),
            pl.BlockSpec((H1, NP), lambda i: (0, 0)),
            pl.BlockSpec((H1, NP), lambda i: (0, 0)),
        ],
        out_shape=[
            jax.ShapeDtypeStruct((H1, NP, W1AUG), jnp.float32),
            jax.ShapeDtypeStruct((H1, NP), jnp.float32),
            jax.ShapeDtypeStruct((H1, NP), jnp.float32),
        ],
    )(x, w1, att_src1, att_dst1)


_MESH = plsc.VectorSubcoreMesh(core_axis_name="c", subcore_axis_name="s")
_SC_PARAMS = pltpu.CompilerParams(needs_layout_passes=False,
                                  use_tc_tiling_on_sc=False)


@functools.partial(
    pl.kernel,
    out_type=jax.ShapeDtypeStruct((H1 * NP, W1AUG), jnp.float32),
    mesh=_MESH,
    scratch_types=[
        pltpu.VMEM((E1T,), jnp.int32),
        pltpu.VMEM((E1T,), jnp.int32),
        pltpu.VMEM((N,), jnp.float32),
        pltpu.VMEM((N,), jnp.float32),
        pltpu.VMEM((K,), jnp.int32),
        pltpu.VMEM((K,), jnp.int32),
        pltpu.VMEM((K,), jnp.float32),
        pltpu.VMEM((K, W1AUG), jnp.float32),
        pltpu.VMEM((128, W1AUG), jnp.float32),
        pltpu.VMEM_SHARED((NP, W1AUG), jnp.float32),
        pltpu.SemaphoreType.DMA,
    ],
    compiler_params=_SC_PARAMS,
)
def _kb(haug_hbm, asrc_hbm, adst_hbm, src_hbm, dst_hbm, acc_hbm,
        src_v, dst_v, as_v, ad_v, gidx, didx, w_v, rows, zbuf, acc_sp, sem):
    c = lax.axis_index("c")
    s = lax.axis_index("s")
    ebase = s * E1T
    pltpu.sync_copy(src_hbm.at[pl.ds(ebase, E1T)], src_v)
    pltpu.sync_copy(dst_hbm.at[pl.ds(ebase, E1T)], dst_v)

    z16 = jnp.zeros((16,), jnp.float32)

    def zb_body(r, carry):
        for q in range(W1AUG // 16):
            zbuf[r, pl.ds(q * 16, 16)] = z16
        return carry

    lax.fori_loop(0, 128, zb_body, 0)
    rbase = s * RPT

    def head_body(jh, carry):
        h = c * (H1 // NC) + jh
        hoff = h * NP
        pltpu.sync_copy(asrc_hbm.at[pl.ds(hoff, N)], as_v)
        pltpu.sync_copy(adst_hbm.at[pl.ds(hoff, N)], ad_v)
        for z in range(RPT // 128):
            pltpu.sync_copy(zbuf, acc_sp.at[pl.ds(rbase + z * 128, 128)])
        plsc.subcore_barrier()

        def chunk_body(ch, carry2):
            off = ch * K
            for i in range(K // 16):
                sl = pl.ds(off + i * 16, 16)
                sidx = src_v[sl]
                didx16 = dst_v[sl]
                a_s = plsc.load_gather(as_v, [sidx])
                a_d = plsc.load_gather(ad_v, [didx16])
                e = a_s + a_d
                e = jnp.maximum(e, 0.2 * e)
                w_v[pl.ds(i * 16, 16)] = jnp.exp(e)
                gidx[pl.ds(i * 16, 16)] = sidx + hoff
                didx[pl.ds(i * 16, 16)] = didx16
            pltpu.async_copy(haug_hbm.at[gidx], rows, sem).wait()

            def mul_body(k, carry3):
                wk = plsc.load_gather(w_v, [jnp.zeros((16,), jnp.int32) + k])
                for q in range(W1AUG // 16):
                    sl2 = pl.ds(q * 16, 16)
                    rows[k, sl2] = rows[k, sl2] * wk
                return carry3

            lax.fori_loop(0, K, mul_body, 0)
            pltpu.sync_copy(rows, acc_sp.at[didx], add=True)
            return carry2

        lax.fori_loop(0, NCH1, chunk_body, 0)
        plsc.subcore_barrier()
        pltpu.sync_copy(acc_sp.at[pl.ds(rbase, RPT)],
                        acc_hbm.at[pl.ds(hoff + rbase, RPT)])
        return carry

    lax.fori_loop(0, H1 // NC, head_body, 0)


def _kc_body(acc_ref, b1_ref, w2_ref, as2_ref, ad2_ref, h2aug_ref, a2_ref):
    i = pl.program_id(0)
    h2 = jnp.zeros((BN, C2), jnp.float32)
    for hh in range(H1):
        num = acc_ref[hh, :, 0:HID]
        den = acc_ref[hh, :, HID:HID + 1]
        t = num / (den + 1e-16) + b1_ref[pl.ds(hh * HID, HID)][None, :]
        t = jnp.where(t > 0, t, jnp.exp(jnp.minimum(t, 0.0)) - 1.0)
        h2 = h2 + jnp.dot(t, w2_ref[pl.ds(hh * HID, HID), :],
                          preferred_element_type=jnp.float32)
    ones = jnp.ones((BN, 1), jnp.float32)
    pad = jnp.zeros((BN, W2AUG - C2 - 1), jnp.float32)
    h2aug_ref[...] = jnp.concatenate([h2, ones, pad], axis=1)
    col = pl.ds(i * BN, BN)
    a2_ref[pl.ds(0, 1), col] = jnp.sum(h2 * as2_ref[0][None, :], axis=1)[None, :]
    a2_ref[pl.ds(1, 1), col] = jnp.sum(h2 * ad2_ref[0][None, :], axis=1)[None, :]


def _stage_c(acc1, b1, w2, att_src2, att_dst2):
    return pl.pallas_call(
        _kc_body,
        grid=(NB,),
        in_specs=[
            pl.BlockSpec((H1, BN, W1AUG), lambda i: (0, i, 0)),
            pl.BlockSpec((H1 * HID,), lambda i: (0,)),
            pl.BlockSpec((H1 * HID, C2), lambda i: (0, 0)),
            pl.BlockSpec((1, C2), lambda i: (0, 0)),
            pl.BlockSpec((1, C2), lambda i: (0, 0)),
        ],
        out_specs=[
            pl.BlockSpec((BN, W2AUG), lambda i: (i, 0)),
            pl.BlockSpec((2, NP), lambda i: (0, 0)),
        ],
        out_shape=[
            jax.ShapeDtypeStruct((NP, W2AUG), jnp.float32),
            jax.ShapeDtypeStruct((2, NP), jnp.float32),
        ],
    )(acc1, b1, w2, att_src2, att_dst2)


@functools.partial(
    pl.kernel,
    out_type=jax.ShapeDtypeStruct((NC * NP, W2AUG), jnp.float32),
    mesh=_MESH,
    scratch_types=[
        pltpu.VMEM((E2T,), jnp.int32),
        pltpu.VMEM((E2T,), jnp.int32),
        pltpu.VMEM((N,), jnp.float32),
        pltpu.VMEM((N,), jnp.float32),
        pltpu.VMEM((K,), jnp.int32),
        pltpu.VMEM((K,), jnp.int32),
        pltpu.VMEM((K,), jnp.float32),
        pltpu.VMEM((K, W2AUG), jnp.float32),
        pltpu.VMEM((RPT, W2AUG), jnp.float32),
        pltpu.VMEM_SHARED((NP, W2AUG), jnp.float32),
        pltpu.SemaphoreType.DMA,
    ],
    compiler_params=_SC_PARAMS,
)
def _kd(h2aug_hbm, a2_hbm, src_hbm, dst_hbm, accp_hbm,
        src_v, dst_v, as_v, ad_v, gidx, didx, w_v, rows, zbuf, acc_sp, sem):
    c = lax.axis_index("c")
    s = lax.axis_index("s")
    ebase = (c * NS + s) * E2T
    pltpu.sync_copy(src_hbm.at[pl.ds(ebase, E2T)], src_v)
    pltpu.sync_copy(dst_hbm.at[pl.ds(ebase, E2T)], dst_v)
    pltpu.sync_copy(a2_hbm.at[pl.ds(0, N)], as_v)
    pltpu.sync_copy(a2_hbm.at[pl.ds(NP, N)], ad_v)

    z16 = jnp.zeros((16,), jnp.float32)

    def zb_body(r, carry):
        zbuf[r, pl.ds(0, 16)] = z16
        return carry

    lax.fori_loop(0, RPT, zb_body, 0)
    rbase = s * RPT
    pltpu.sync_copy(zbuf, acc_sp.at[pl.ds(rbase, RPT)])
    plsc.subcore_barrier()

    def chunk_body(ch, carry2):
        off = ch * K
        for i in range(K // 16):
            sl = pl.ds(off + i * 16, 16)
            sidx = src_v[sl]
            didx16 = dst_v[sl]
            a_s = plsc.load_gather(as_v, [sidx])
            a_d = plsc.load_gather(ad_v, [didx16])
            e = a_s + a_d
            e = jnp.maximum(e, 0.2 * e)
            w_v[pl.ds(i * 16, 16)] = jnp.exp(e)
            gidx[pl.ds(i * 16, 16)] = sidx
            didx[pl.ds(i * 16, 16)] = didx16
        pltpu.async_copy(h2aug_hbm.at[gidx], rows, sem).wait()

        def mul_body(k, carry3):
            wk = plsc.load_gather(w_v, [jnp.zeros((16,), jnp.int32) + k])
            rows[k, pl.ds(0, 16)] = rows[k, pl.ds(0, 16)] * wk
            return carry3

        lax.fori_loop(0, K, mul_body, 0)
        pltpu.sync_copy(rows, acc_sp.at[didx], add=True)
        return carry2

    lax.fori_loop(0, NCH2, chunk_body, 0)
    plsc.subcore_barrier()
    pltpu.sync_copy(acc_sp.at[pl.ds(rbase, RPT)],
                    accp_hbm.at[pl.ds(c * NP + rbase, RPT)])


def _ke_body(accp_ref, b2_ref, out_ref):
    ssum = accp_ref[0] + accp_ref[1]
    out_ref[...] = (ssum[:, 0:C2] / (ssum[:, C2:C2 + 1] + 1e-16)
                    + b2_ref[...][None, :])


def _stage_e(accp, b2):
    return pl.pallas_call(
        _ke_body,
        grid=(NB,),
        in_specs=[
            pl.BlockSpec((2, BN, W2AUG), lambda i: (0, i, 0)),
            pl.BlockSpec((C2,), lambda i: (0,)),
        ],
        out_specs=pl.BlockSpec((BN, C2), lambda i: (i, 0)),
        out_shape=jax.ShapeDtypeStruct((N, C2), jnp.float32),
    )(accp, b2)


def kernel(x, edge_index, W1, att_src1, att_dst1, b1,
           W2, att_src2, att_dst2, b2):
    src = edge_index[0].astype(jnp.int32)
    dst = edge_index[1].astype(jnp.int32)
    x_p = jnp.concatenate(
        [x, jnp.zeros((NP - N, D_IN), jnp.float32)], axis=0)
    haug, asrc, adst = _stage_a(x_p, W1, att_src1, att_dst1)
    acc1 = _kb(haug.reshape(H1 * NP, W1AUG), asrc.reshape(H1 * NP),
               adst.reshape(H1 * NP), src, dst)
    h2aug, a2 = _stage_c(acc1.reshape(H1, NP, W1AUG), b1, W2,
                         att_src2, att_dst2)
    accp = _kd(h2aug, a2.reshape(2 * NP), src, dst)
    return _stage_e(accp.reshape(NC, NP, W2AUG), b2)
